# Initial kernel scaffold; baseline (speedup 1.0000x reference)
import jax
import jax.numpy as jnp
from jax.experimental import pallas as pl


def kernel(x, edge_index, edge_attr, Wk, bk, Wq, bq, Wv, bv, Wskip, bias):
    raise NotImplementedError("write your pallas kernel here")


# speedup vs baseline: 3.0445x; 3.0445x over previous
import jax
import jax.numpy as jnp
from jax import lax
from jax.experimental import pallas as pl
from jax.experimental.pallas import tpu as pltpu
from jax.experimental.pallas import tpu_sc as plsc

N = 10000
E = 320000
D = 128
DE = 16

NC = 2
NS = 16
L = 16
NW = NC * NS
BE = 64
EPW = 10048
EP = EPW * NW
NIT = EPW // BE
NPAD = 10240
RPT = NPAD // NS
RCH = BE
NCH = RPT // RCH

BN = 1024
BNC = 1000
BEB = 5024


def _proj_body(x_ref, w_ref, b_ref, xk_ref, xq_ref, xv_ref, sk_ref):
    acc = jnp.dot(x_ref[...], w_ref[...], preferred_element_type=jnp.float32)
    xk_ref[...] = acc[:, 0:D]
    xq_ref[...] = acc[:, D:2 * D]
    xv_ref[...] = acc[:, 2 * D:3 * D]
    sk_ref[...] = acc[:, 3 * D:4 * D] + b_ref[...]


def _edge_body(ea_ref, wg_ref, wv_ref, bg_ref, bv_ref, eg_ref, ev_ref):
    ea = ea_ref[...]
    eg_ref[...] = jnp.dot(ea, wg_ref[...], preferred_element_type=jnp.float32) + bg_ref[...]
    ev_ref[...] = jnp.dot(ea, wv_ref[...], preferred_element_type=jnp.float32) + bv_ref[...]


def _comb_body(p_ref, s_ref, o_ref):
    o_ref[...] = p_ref[0] + p_ref[1] + s_ref[...]


def _sc_body(xk_hbm, xq_hbm, xv_hbm, eg_hbm, ev_hbm, src_hbm, dst_hbm, out_hbm,
             didx, sidx, kbuf, qbuf, v2buf, zbuf, vbuf, shared,
             sem_idx, sem):
    core = lax.axis_index("c")
    sid = lax.axis_index("s")
    wid = core * NS + sid

    def zrow(r, carry):
        for c in range(D // L):
            kbuf[r, pl.ds(c * L, L)] = jnp.zeros((L,), jnp.float32)
        return carry

    lax.fori_loop(0, RCH, zrow, 0)
    r0 = sid * RPT
    for j in range(NCH):
        pltpu.sync_copy(kbuf, shared.at[pl.ds(r0 + j * RCH, RCH)])
    plsc.subcore_barrier()

    ebase = wid * EPW

    def it_body(i, carry):
        e0 = ebase + i * BE
        cpd = pltpu.async_copy(dst_hbm.at[pl.ds(e0, BE)], didx, sem_idx)
        cps = pltpu.async_copy(src_hbm.at[pl.ds(e0, BE)], sidx, sem_idx)
        cpg = pltpu.async_copy(eg_hbm.at[pl.ds(e0, BE)], zbuf, sem)
        cpv = pltpu.async_copy(ev_hbm.at[pl.ds(e0, BE)], vbuf, sem)
        cpd.wait()
        cps.wait()
        cpk = pltpu.async_copy(xk_hbm.at[didx], kbuf, sem)
        cpq = pltpu.async_copy(xq_hbm.at[sidx], qbuf, sem)
        cp2 = pltpu.async_copy(xv_hbm.at[sidx], v2buf, sem)
        cpg.wait()
        cpv.wait()
        cpk.wait()
        cpq.wait()
        cp2.wait()

        def edge(e, c2):
            for c in range(D // L):
                sl = pl.ds(c * L, L)
                z = kbuf[e, sl] + qbuf[e, sl] + zbuf[e, sl]
                s = 1.0 / (1.0 + jnp.exp(-z))
                zbuf[e, sl] = s * (v2buf[e, sl] + vbuf[e, sl])
            return c2

        lax.fori_loop(0, BE, edge, 0)
        pltpu.sync_copy(zbuf, shared.at[didx], add=True)
        return carry

    lax.fori_loop(0, NIT, it_body, 0)
    plsc.subcore_barrier()

    for j in range(NCH):
        rr = pl.ds(r0 + j * RCH, RCH)
        pltpu.sync_copy(shared.at[rr], kbuf)
        pltpu.sync_copy(kbuf, out_hbm.at[core, rr])


_sc_call = pl.kernel(
    _sc_body,
    out_type=jax.ShapeDtypeStruct((NC, NPAD, D), jnp.float32),
    mesh=plsc.VectorSubcoreMesh(core_axis_name="c", subcore_axis_name="s"),
    scratch_types=[
        pltpu.VMEM((BE,), jnp.int32),
        pltpu.VMEM((BE,), jnp.int32),
        pltpu.VMEM((BE, D), jnp.float32),
        pltpu.VMEM((BE, D), jnp.float32),
        pltpu.VMEM((BE, D), jnp.float32),
        pltpu.VMEM((BE, D), jnp.float32),
        pltpu.VMEM((BE, D), jnp.float32),
        pltpu.VMEM_SHARED((NPAD, D), jnp.float32),
        pltpu.SemaphoreType.DMA,
        pltpu.SemaphoreType.DMA,
    ],
)


def kernel(x, edge_index, edge_attr, Wk, bk, Wq, bq, Wv, bv, Wskip, bias):
    x_pad = jnp.pad(x, ((0, NPAD - N), (0, 0)))
    w_all = jnp.concatenate([Wk[:D], Wq[:D], Wv[:D], Wskip], axis=1)
    xk, xq, xv, skip = pl.pallas_call(
        _proj_body,
        grid=(NPAD // BN,),
        in_specs=[
            pl.BlockSpec((BN, D), lambda i: (i, 0)),
            pl.BlockSpec((D, 4 * D), lambda i: (0, 0)),
            pl.BlockSpec((1, D), lambda i: (0, 0)),
        ],
        out_specs=[pl.BlockSpec((BN, D), lambda i: (i, 0))] * 4,
        out_shape=[jax.ShapeDtypeStruct((NPAD, D), jnp.float32)] * 4,
    )(x_pad, w_all, bias.reshape(1, D))

    ea_pad = jnp.pad(edge_attr, ((0, EP - E), (0, 0)))
    src_pad = jnp.pad(edge_index[0], (0, EP - E), constant_values=N)
    dst_pad = jnp.pad(edge_index[1], (0, EP - E), constant_values=N)
    wg = Wk[D:] + Wq[D:]
    eg, ev = pl.pallas_call(
        _edge_body,
        grid=(EP // BEB,),
        in_specs=[
            pl.BlockSpec((BEB, DE), lambda i: (i, 0)),
            pl.BlockSpec((DE, D), lambda i: (0, 0)),
            pl.BlockSpec((DE, D), lambda i: (0, 0)),
            pl.BlockSpec((1, D), lambda i: (0, 0)),
            pl.BlockSpec((1, D), lambda i: (0, 0)),
        ],
        out_specs=[pl.BlockSpec((BEB, D), lambda i: (i, 0))] * 2,
        out_shape=[jax.ShapeDtypeStruct((EP, D), jnp.float32)] * 2,
    )(ea_pad, wg, Wv[D:], (bk + bq).reshape(1, D), bv.reshape(1, D))

    partial = _sc_call(xk, xq, xv, eg, ev, src_pad, dst_pad)

    out = pl.pallas_call(
        _comb_body,
        grid=(N // BNC,),
        in_specs=[
            pl.BlockSpec((NC, BNC, D), lambda i: (0, i, 0)),
            pl.BlockSpec((BNC, D), lambda i: (i, 0)),
        ],
        out_specs=pl.BlockSpec((BNC, D), lambda i: (i, 0)),
        out_shape=jax.ShapeDtypeStruct((N, D), jnp.float32),
    )(partial, skip)
    return out
